# baseline (speedup 1.0000x reference)
import jax
import jax.numpy as jnp
from jax.experimental import pallas as pl
from jax.experimental.pallas import tpu as pltpu

NFEAT = 256
NHID = 32
NHEADS = 4
ALPHA = 0.2
N = 1024

BLK = 1024
NB = N // BLK
SLOT = 128
CLIP = 60.0

_F32 = jnp.float32
_BF16 = jnp.bfloat16


def _elu(x):
    return jnp.where(x > 0, x, jnp.exp(jnp.minimum(x, 0.0)) - 1.0)


def _cexp(z):
    return jnp.exp(jnp.clip(z, -CLIP, CLIP))


def _gat_body(x_ref, ws_ref, a1_ref, a2_ref, adj_ref, wout_ref, aout_ref,
              out_ref, haug_ref, e1_ref, e1a_ref, e2t_ref, e2ta_ref,
              hl2_ref, g1_ref, g1a_ref, g2t_ref, g2ta_ref, haug2_ref,
              adjv_ref, adj_sem):
    i = pl.program_id(0)

    @pl.when(i == 0)
    def _init1():
        adj_copy = pltpu.make_async_copy(adj_ref, adjv_ref, adj_sem)
        adj_copy.start()
        ones_col = jnp.ones((N, 1), dtype=_F32)
        for k in range(NHEADS):
            hk = jnp.dot(x_ref[...], ws_ref[k],
                         preferred_element_type=_F32)
            haug_ref[:, k * SLOT:k * SLOT + NHID] = hk
            haug_ref[:, k * SLOT + NHID:k * SLOT + NHID + 1] = ones_col
            f1n = jnp.dot(hk, -a1_ref[:, k:k + 1],
                          preferred_element_type=_F32)
            f2tn = jax.lax.dot_general(
                -a2_ref[:, k:k + 1], hk,
                dimension_numbers=(((0,), (1,)), ((), ())),
                preferred_element_type=_F32)
            e1_ref[:, k:k + 1] = _cexp(f1n)
            e1a_ref[:, k:k + 1] = _cexp(ALPHA * f1n)
            e2t_ref[k:k + 1, :] = _cexp(f2tn)
            e2ta_ref[k:k + 1, :] = _cexp(ALPHA * f2tn)
        adj_copy.wait()

    @pl.when(i == NB)
    def _init2():
        h2 = jnp.dot(hl2_ref[...], wout_ref[...],
                     preferred_element_type=_F32)
        h2t = jax.lax.dot_general(
            wout_ref[...], hl2_ref[...],
            dimension_numbers=(((0,), (1,)), ((), ())),
            preferred_element_type=_F32)
        z1 = h2 * (-aout_ref[0:1, 0:1])
        z2 = h2t * (-aout_ref[0:1, 1:2])
        g1_ref[...] = _cexp(z1)
        g1a_ref[...] = _cexp(ALPHA * z1)
        g2t_ref[...] = _cexp(z2)
        g2ta_ref[...] = _cexp(ALPHA * z2)
        haug2_ref[:, 0:1] = h2
        haug2_ref[:, 1:2] = jnp.ones((N, 1), dtype=_F32)

    ib = jnp.where(i < NB, i, i - NB)
    r0 = ib * BLK
    adj_blk = adjv_ref[pl.ds(r0, BLK), :].astype(_F32)

    @pl.when(i < NB)
    def _layer1():
        for k in range(NHEADS):
            p1 = e1_ref[pl.ds(r0, BLK), k:k + 1] * e2t_ref[k:k + 1, :]
            p2 = e1a_ref[pl.ds(r0, BLK), k:k + 1] * e2ta_ref[k:k + 1, :]
            e = jnp.minimum(p1, p2) * adj_blk
            ns = jnp.dot(e, haug_ref[:, k * SLOT:(k + 1) * SLOT],
                         preferred_element_type=_F32)
            hp = ns[:, :NHID] / ns[:, NHID:NHID + 1]
            hl2_ref[pl.ds(r0, BLK), k * NHID:(k + 1) * NHID] = _elu(hp)

    @pl.when(i >= NB)
    def _layer2():
        p1 = g1_ref[pl.ds(r0, BLK), :] * g2t_ref[...]
        p2 = g1a_ref[pl.ds(r0, BLK), :] * g2ta_ref[...]
        e = jnp.minimum(p1, p2) * adj_blk
        ns = jnp.dot(e, haug2_ref[...],
                     preferred_element_type=_F32)
        out_ref[...] = jax.nn.sigmoid(_elu(ns[:, 0:1] / ns[:, 1:2]))


@jax.jit
def kernel(x, adj, Ws, attn_a, W_out, a_out):
    a1s = jnp.transpose(attn_a[:, 0, :NHID])
    a2s = jnp.transpose(attn_a[:, 0, NHID:])
    adjh = adj.astype(_BF16)

    out = pl.pallas_call(
        _gat_body,
        grid=(2 * NB,),
        in_specs=[
            pl.BlockSpec((N, NFEAT), lambda i: (0, 0)),
            pl.BlockSpec((NHEADS, NFEAT, NHID), lambda i: (0, 0, 0)),
            pl.BlockSpec((NHID, NHEADS), lambda i: (0, 0)),
            pl.BlockSpec((NHID, NHEADS), lambda i: (0, 0)),
            pl.BlockSpec(memory_space=pl.ANY),
            pl.BlockSpec((NHEADS * NHID, 1), lambda i: (0, 0)),
            pl.BlockSpec((1, 2), lambda i: (0, 0)),
        ],
        out_specs=pl.BlockSpec((BLK, 1), lambda i: (jnp.maximum(i - NB, 0), 0)),
        out_shape=jax.ShapeDtypeStruct((N, 1), _F32),
        scratch_shapes=[
            pltpu.VMEM((N, NHEADS * SLOT), _F32),
            pltpu.VMEM((N, NHEADS), _F32),
            pltpu.VMEM((N, NHEADS), _F32),
            pltpu.VMEM((NHEADS, N), _F32),
            pltpu.VMEM((NHEADS, N), _F32),
            pltpu.VMEM((N, NHEADS * NHID), _F32),
            pltpu.VMEM((N, 1), _F32),
            pltpu.VMEM((N, 1), _F32),
            pltpu.VMEM((1, N), _F32),
            pltpu.VMEM((1, N), _F32),
            pltpu.VMEM((N, SLOT), _F32),
            pltpu.VMEM((N, N), _BF16),
            pltpu.SemaphoreType.DMA,
        ],
    )(x, Ws, a1s, a2s, adjh, W_out, a_out)

    return out


# speedup vs baseline: 1.2020x; 1.2020x over previous
import jax
import jax.numpy as jnp
from jax.experimental import pallas as pl
from jax.experimental.pallas import tpu as pltpu

NFEAT = 256
NHID = 32
NHEADS = 4
ALPHA = 0.2
N = 1024

BLK = 1024
NB = N // BLK
SLOT = 128
CLIP = 60.0

_F32 = jnp.float32
_BF16 = jnp.bfloat16


def _elu(x):
    return jnp.where(x > 0, x, jnp.exp(jnp.minimum(x, 0.0)) - 1.0)


def _cexp(z):
    return jnp.exp(jnp.clip(z, -CLIP, CLIP))


def _gat_body(x_ref, ws_ref, a1_ref, a2_ref, adj_ref, wout_ref, aout_ref,
              out_ref, haug_ref, e1_ref, e1a_ref, e2t_ref, e2ta_ref,
              hl2_ref, g1_ref, g1a_ref, g2t_ref, g2ta_ref, haug2_ref,
              adjv_ref, adj_sem):
    i = pl.program_id(0)

    @pl.when(i == 0)
    def _init1():
        adj_copy = pltpu.make_async_copy(adj_ref, adjv_ref, adj_sem)
        adj_copy.start()
        ones_col = jnp.ones((N, 1), dtype=_F32)
        for k in range(NHEADS):
            hk = jnp.dot(x_ref[...], ws_ref[k],
                         preferred_element_type=_F32)
            haug_ref[:, k * SLOT:k * SLOT + NHID] = hk
            haug_ref[:, k * SLOT + NHID:k * SLOT + NHID + 1] = ones_col
            f1n = jnp.dot(hk, -a1_ref[:, k:k + 1],
                          preferred_element_type=_F32)
            f2tn = jax.lax.dot_general(
                -a2_ref[:, k:k + 1], hk,
                dimension_numbers=(((0,), (1,)), ((), ())),
                preferred_element_type=_F32)
            e1_ref[:, k:k + 1] = _cexp(f1n)
            e1a_ref[:, k:k + 1] = _cexp(ALPHA * f1n)
            e2t_ref[k:k + 1, :] = _cexp(f2tn)
            e2ta_ref[k:k + 1, :] = _cexp(ALPHA * f2tn)
        adj_copy.wait()

    @pl.when(i == NB)
    def _init2():
        h2 = jnp.dot(hl2_ref[...], wout_ref[...],
                     preferred_element_type=_F32)
        h2t = jax.lax.dot_general(
            wout_ref[...], hl2_ref[...],
            dimension_numbers=(((0,), (1,)), ((), ())),
            preferred_element_type=_F32)
        z1 = h2 * (-aout_ref[0:1, 0:1])
        z2 = h2t * (-aout_ref[0:1, 1:2])
        g1_ref[...] = _cexp(z1)
        g1a_ref[...] = _cexp(ALPHA * z1)
        g2t_ref[...] = _cexp(z2)
        g2ta_ref[...] = _cexp(ALPHA * z2)
        haug2_ref[:, 0:1] = h2
        haug2_ref[:, 1:2] = jnp.ones((N, 1), dtype=_F32)

    ib = jnp.where(i < NB, i, i - NB)
    r0 = ib * BLK
    adj_blk = adjv_ref[pl.ds(r0, BLK), :]

    @pl.when(i < NB)
    def _layer1():
        for k in range(NHEADS):
            p1 = e1_ref[pl.ds(r0, BLK), k:k + 1] * e2t_ref[k:k + 1, :]
            p2 = e1a_ref[pl.ds(r0, BLK), k:k + 1] * e2ta_ref[k:k + 1, :]
            e = jnp.minimum(p1, p2) * adj_blk
            ns = jnp.dot(e, haug_ref[:, k * SLOT:(k + 1) * SLOT],
                         preferred_element_type=_F32)
            hp = ns[:, :NHID] / ns[:, NHID:NHID + 1]
            hl2_ref[pl.ds(r0, BLK), k * NHID:(k + 1) * NHID] = _elu(hp)

    @pl.when(i >= NB)
    def _layer2():
        p1 = g1_ref[pl.ds(r0, BLK), :] * g2t_ref[...]
        p2 = g1a_ref[pl.ds(r0, BLK), :] * g2ta_ref[...]
        e = jnp.minimum(p1, p2) * adj_blk
        ns = jnp.dot(e, haug2_ref[...],
                     preferred_element_type=_F32)
        out_ref[...] = jax.nn.sigmoid(_elu(ns[:, 0:1] / ns[:, 1:2]))


@jax.jit
def kernel(x, adj, Ws, attn_a, W_out, a_out):
    a1s = jnp.transpose(attn_a[:, 0, :NHID])
    a2s = jnp.transpose(attn_a[:, 0, NHID:])

    out = pl.pallas_call(
        _gat_body,
        grid=(2 * NB,),
        in_specs=[
            pl.BlockSpec((N, NFEAT), lambda i: (0, 0)),
            pl.BlockSpec((NHEADS, NFEAT, NHID), lambda i: (0, 0, 0)),
            pl.BlockSpec((NHID, NHEADS), lambda i: (0, 0)),
            pl.BlockSpec((NHID, NHEADS), lambda i: (0, 0)),
            pl.BlockSpec(memory_space=pl.ANY),
            pl.BlockSpec((NHEADS * NHID, 1), lambda i: (0, 0)),
            pl.BlockSpec((1, 2), lambda i: (0, 0)),
        ],
        out_specs=pl.BlockSpec((BLK, 1), lambda i: (jnp.maximum(i - NB, 0), 0)),
        out_shape=jax.ShapeDtypeStruct((N, 1), _F32),
        scratch_shapes=[
            pltpu.VMEM((N, NHEADS * SLOT), _F32),
            pltpu.VMEM((N, NHEADS), _F32),
            pltpu.VMEM((N, NHEADS), _F32),
            pltpu.VMEM((NHEADS, N), _F32),
            pltpu.VMEM((NHEADS, N), _F32),
            pltpu.VMEM((N, NHEADS * NHID), _F32),
            pltpu.VMEM((N, 1), _F32),
            pltpu.VMEM((N, 1), _F32),
            pltpu.VMEM((1, N), _F32),
            pltpu.VMEM((1, N), _F32),
            pltpu.VMEM((N, SLOT), _F32),
            pltpu.VMEM((N, N), _F32),
            pltpu.SemaphoreType.DMA,
        ],
    )(x, Ws, a1s, a2s, adj, W_out, a_out)

    return out


# speedup vs baseline: 1.2598x; 1.0482x over previous
import jax
import jax.numpy as jnp
from jax.experimental import pallas as pl
from jax.experimental.pallas import tpu as pltpu

NFEAT = 256
NHID = 32
NHEADS = 4
ALPHA = 0.2
N = 1024

CLIP = 60.0

_F32 = jnp.float32


def _elu(x):
    return jnp.where(x > 0, x, jnp.exp(jnp.minimum(x, 0.0)) - 1.0)


def _cexp(z):
    return jnp.exp(jnp.clip(z, -CLIP, CLIP))


def _gat_body(x_ref, ws_ref, a1_ref, a2_ref, adj_ref, wout_ref, aout_ref,
              out_ref, adjv_ref, adj_sem):
    adj_copy = pltpu.make_async_copy(adj_ref, adjv_ref, adj_sem)
    adj_copy.start()

    ones_col = jnp.ones((N, 1), dtype=_F32)
    x = x_ref[...]

    hs, e1s, e1as, e2ts, e2tas = [], [], [], [], []
    for k in range(NHEADS):
        hk = jnp.dot(x, ws_ref[k], preferred_element_type=_F32)
        f1n = jnp.dot(hk, -a1_ref[:, k:k + 1],
                      preferred_element_type=_F32)
        f2tn = jax.lax.dot_general(
            -a2_ref[:, k:k + 1], hk,
            dimension_numbers=(((0,), (1,)), ((), ())),
            preferred_element_type=_F32)
        hs.append(jnp.concatenate([hk, ones_col], axis=1))
        e1s.append(_cexp(f1n))
        e1as.append(_cexp(ALPHA * f1n))
        e2ts.append(_cexp(f2tn))
        e2tas.append(_cexp(ALPHA * f2tn))

    adj_copy.wait()
    adj = adjv_ref[...]

    hl2_parts = []
    for k in range(NHEADS):
        p1 = e1s[k] * e2ts[k]
        p2 = e1as[k] * e2tas[k]
        e = jnp.minimum(p1, p2) * adj
        ns = jnp.dot(e, hs[k], preferred_element_type=_F32)
        hp = ns[:, :NHID] / ns[:, NHID:NHID + 1]
        hl2_parts.append(_elu(hp))
    hl2 = jnp.concatenate(hl2_parts, axis=1)

    h2 = jnp.dot(hl2, wout_ref[...], preferred_element_type=_F32)
    h2t = jax.lax.dot_general(
        wout_ref[...], hl2,
        dimension_numbers=(((0,), (1,)), ((), ())),
        preferred_element_type=_F32)
    z1 = h2 * (-aout_ref[0:1, 0:1])
    z2 = h2t * (-aout_ref[0:1, 1:2])
    p1 = _cexp(z1) * _cexp(z2)
    p2 = _cexp(ALPHA * z1) * _cexp(ALPHA * z2)
    e = jnp.minimum(p1, p2) * adj
    ns = jnp.dot(e, jnp.concatenate([h2, ones_col], axis=1),
                 preferred_element_type=_F32)
    out_ref[...] = jax.nn.sigmoid(_elu(ns[:, 0:1] / ns[:, 1:2]))


@jax.jit
def kernel(x, adj, Ws, attn_a, W_out, a_out):
    a1s = jnp.transpose(attn_a[:, 0, :NHID])
    a2s = jnp.transpose(attn_a[:, 0, NHID:])

    out = pl.pallas_call(
        _gat_body,
        grid=(1,),
        in_specs=[
            pl.BlockSpec((N, NFEAT), lambda i: (0, 0)),
            pl.BlockSpec((NHEADS, NFEAT, NHID), lambda i: (0, 0, 0)),
            pl.BlockSpec((NHID, NHEADS), lambda i: (0, 0)),
            pl.BlockSpec((NHID, NHEADS), lambda i: (0, 0)),
            pl.BlockSpec(memory_space=pl.ANY),
            pl.BlockSpec((NHEADS * NHID, 1), lambda i: (0, 0)),
            pl.BlockSpec((1, 2), lambda i: (0, 0)),
        ],
        out_specs=pl.BlockSpec((N, 1), lambda i: (0, 0)),
        out_shape=jax.ShapeDtypeStruct((N, 1), _F32),
        scratch_shapes=[
            pltpu.VMEM((N, N), _F32),
            pltpu.SemaphoreType.DMA,
        ],
    )(x, Ws, a1s, a2s, adj, W_out, a_out)

    return out
